# Initial kernel scaffold; baseline (speedup 1.0000x reference)
import functools

import jax
import jax.numpy as jnp
import numpy as np
from jax import lax
from jax.experimental import pallas as pl
from jax.experimental.pallas import tpu as pltpu
from jax.experimental.pallas import tpu_sc as plsc

N_NODES = 10000
N_EDGES = 160000
F = 16
N_GRAPHS = 64

NC = 2
NS = 16
NW = NC * NS
EDGES_PER_TILE = N_EDGES // NW
NODE_ROWS_PER_TILE = N_NODES // NS
SCATTER_CHUNK = 1000

_MESH = plsc.VectorSubcoreMesh(core_axis_name="c", subcore_axis_name="s")

_REP_H = jnp.asarray(np.kron(np.eye(F), np.ones((1, F))), jnp.float32)
_REP_X = jnp.asarray(np.kron(np.ones((1, F)), np.eye(F)), jnp.float32)


def _leaky(v):
    return jnp.where(v >= 0, v, 0.01 * v)


@functools.partial(
    pl.kernel,
    mesh=_MESH,
    out_type=jax.ShapeDtypeStruct((N_EDGES, F), jnp.float32),
    scratch_types=[
        pltpu.VMEM((EDGES_PER_TILE,), jnp.int32),
        pltpu.VMEM((EDGES_PER_TILE, F), jnp.float32),
        pltpu.SemaphoreType.DMA,
    ],
)
def _sc_gather_rows(table_hbm, idx_hbm, out_hbm, idx_v, rows_v, sem):
    wid = lax.axis_index("s") * NC + lax.axis_index("c")
    base = wid * EDGES_PER_TILE
    pltpu.sync_copy(idx_hbm.at[pl.ds(base, EDGES_PER_TILE)], idx_v)
    pltpu.async_copy(table_hbm.at[idx_v], rows_v, sem).wait()
    pltpu.sync_copy(rows_v, out_hbm.at[pl.ds(base, EDGES_PER_TILE)])


def _make_sc_scatter(width):
    nchunks = EDGES_PER_TILE // SCATTER_CHUNK

    @functools.partial(
        pl.kernel,
        mesh=_MESH,
        out_type=jax.ShapeDtypeStruct((NC, N_NODES, width), jnp.float32),
        scratch_types=[
            pltpu.VMEM((SCATTER_CHUNK, width), jnp.float32),
            pltpu.VMEM((SCATTER_CHUNK,), jnp.int32),
            pltpu.VMEM((NODE_ROWS_PER_TILE, width), jnp.float32),
            pltpu.VMEM_SHARED((N_NODES, width), jnp.float32),
        ],
    )
    def _scatter(msg_hbm, dst_hbm, out_hbm, msg_v, idx_v, row_v, acc):
        c = lax.axis_index("c")
        s = lax.axis_index("s")
        base = (c * NS + s) * EDGES_PER_TILE

        def _zero_row(i, carry):
            for j in range(width // 16):
                row_v[i, pl.ds(j * 16, 16)] = jnp.zeros((16,), jnp.float32)
            return carry

        lax.fori_loop(0, NODE_ROWS_PER_TILE, _zero_row, 0)
        pltpu.sync_copy(
            row_v, acc.at[pl.ds(s * NODE_ROWS_PER_TILE, NODE_ROWS_PER_TILE)]
        )
        plsc.subcore_barrier()
        for ch in range(nchunks):
            off = base + ch * SCATTER_CHUNK
            pltpu.sync_copy(msg_hbm.at[pl.ds(off, SCATTER_CHUNK)], msg_v)
            pltpu.sync_copy(dst_hbm.at[pl.ds(off, SCATTER_CHUNK)], idx_v)
            pltpu.sync_copy(msg_v, acc.at[idx_v], add=True)
        plsc.subcore_barrier()
        pltpu.sync_copy(
            acc.at[pl.ds(s * NODE_ROWS_PER_TILE, NODE_ROWS_PER_TILE)], row_v
        )
        pltpu.sync_copy(
            row_v,
            out_hbm.at[c, pl.ds(s * NODE_ROWS_PER_TILE, NODE_ROWS_PER_TILE)],
        )

    return _scatter


_sc_scatter32 = _make_sc_scatter(32)
_sc_scatter16 = _make_sc_scatter(16)


_EDGE_BLOCK = 2000


def _msg_body(pack_counts, ea_ref, xs_ref, w1_ref, b1_ref, rh_ref, rx_ref,
              w2m_ref, b2r_ref, out_ref):
    ea = ea_ref[...]
    xs = xs_ref[...]
    h = jnp.dot(ea, w1_ref[...], preferred_element_type=jnp.float32) + b1_ref[...]
    h = _leaky(h)
    z = (jnp.dot(h, rh_ref[...], preferred_element_type=jnp.float32)
         * jnp.dot(xs, rx_ref[...], preferred_element_type=jnp.float32))
    msg = (jnp.dot(z, w2m_ref[...], preferred_element_type=jnp.float32)
           + jnp.dot(xs, b2r_ref[...], preferred_element_type=jnp.float32))
    if pack_counts:
        out_ref[...] = jnp.concatenate([msg, jnp.ones_like(msg)], axis=1)
    else:
        out_ref[...] = msg


def _tc_messages(edge_attr, xs, W1, b1, W2m, b2r, pack_counts):
    width = 2 * F if pack_counts else F
    grid = N_EDGES // _EDGE_BLOCK
    full = lambda i: (0, 0)
    return pl.pallas_call(
        functools.partial(_msg_body, pack_counts),
        grid=(grid,),
        in_specs=[
            pl.BlockSpec((_EDGE_BLOCK, 4), lambda i: (i, 0)),
            pl.BlockSpec((_EDGE_BLOCK, F), lambda i: (i, 0)),
            pl.BlockSpec((4, F), full),
            pl.BlockSpec((1, F), full),
            pl.BlockSpec((F, F * F), full),
            pl.BlockSpec((F, F * F), full),
            pl.BlockSpec((F * F, F), full),
            pl.BlockSpec((F, F), full),
        ],
        out_specs=pl.BlockSpec((_EDGE_BLOCK, width), lambda i: (i, 0)),
        out_shape=jax.ShapeDtypeStruct((N_EDGES, width), jnp.float32),
    )(edge_attr, xs, W1, b1.reshape(1, F), _REP_H, _REP_X, W2m, b2r)


def _bn1_body(p_ref, bias_ref, g_ref, b_ref, h_ref, cnt_ref):
    p0 = p_ref[0]
    p1 = p_ref[1]
    s = p0[:, :F] + p1[:, :F]
    cnt = jnp.maximum(p0[:, F:] + p1[:, F:], 1.0)
    a = _leaky(s / cnt + bias_ref[...])
    m = jnp.mean(a, axis=0, keepdims=True)
    v = jnp.mean((a - m) ** 2, axis=0, keepdims=True)
    h_ref[...] = g_ref[...] * (a - m) * lax.rsqrt(v + 1e-5) + b_ref[...]
    cnt_ref[...] = cnt


def _tc_bn1(partials, conv_bias, gamma, beta):
    return pl.pallas_call(
        _bn1_body,
        out_shape=(
            jax.ShapeDtypeStruct((N_NODES, F), jnp.float32),
            jax.ShapeDtypeStruct((N_NODES, F), jnp.float32),
        ),
    )(partials, conv_bias.reshape(1, F), gamma.reshape(1, F), beta.reshape(1, F))


def _final_body(p_ref, cnt_ref, bias_ref, g_ref, b_ref, batch_ref, gf_ref,
                w1_ref, b1_ref, w2_ref, b2_ref, out_ref):
    s = p_ref[0] + p_ref[1]
    a = _leaky(s / cnt_ref[...] + bias_ref[...])
    m = jnp.mean(a, axis=0, keepdims=True)
    v = jnp.mean((a - m) ** 2, axis=0, keepdims=True)
    h = g_ref[...] * (a - m) * lax.rsqrt(v + 1e-5) + b_ref[...]
    gid = lax.broadcasted_iota(jnp.int32, (N_GRAPHS, N_NODES), 0)
    oh = (batch_ref[...] == gid).astype(jnp.float32)
    psum = jnp.dot(oh, h, preferred_element_type=jnp.float32)
    gcnt = jnp.maximum(jnp.sum(oh, axis=1, keepdims=True), 1.0)
    g = jnp.concatenate([psum / gcnt, gf_ref[...]], axis=1)
    g = _leaky(jnp.dot(g, w1_ref[...], preferred_element_type=jnp.float32)
               + b1_ref[...])
    g = _leaky(jnp.dot(g, w2_ref[...], preferred_element_type=jnp.float32)
               + b2_ref[...])
    out_ref[...] = g


def _tc_final(partials, cnt, conv_bias, gamma, beta, batch2d, gf,
              lin1_W, lin1_b, lin2_W, lin2_b):
    return pl.pallas_call(
        _final_body,
        out_shape=jax.ShapeDtypeStruct((N_GRAPHS, 32), jnp.float32),
    )(partials, cnt, conv_bias.reshape(1, F), gamma.reshape(1, F),
      beta.reshape(1, F), batch2d, gf, lin1_W, lin1_b.reshape(1, 64),
      lin2_W, lin2_b.reshape(1, 32))


def kernel(x, edge_index, edge_attr, graph_features, batch,
           enn1_W1, enn1_b1, enn1_W2, enn1_b2, conv1_bias, bn1_gamma, bn1_beta,
           enn2_W1, enn2_b1, enn2_W2, enn2_b2, conv2_bias, bn2_gamma, bn2_beta,
           lin1_W, lin1_b, lin2_W, lin2_b):
    src = edge_index[0]
    dst = edge_index[1]
    W2m1 = enn1_W2.reshape(F, F, F).reshape(F * F, F)
    b2r1 = enn1_b2.reshape(F, F)
    W2m2 = enn2_W2.reshape(F, F, F).reshape(F * F, F)
    b2r2 = enn2_b2.reshape(F, F)

    xs1 = _sc_gather_rows(x, src)
    msg1 = _tc_messages(edge_attr, xs1, enn1_W1, enn1_b1, W2m1, b2r1, True)
    part1 = _sc_scatter32(msg1, dst)
    h1, cnt = _tc_bn1(part1, conv1_bias, bn1_gamma, bn1_beta)

    xs2 = _sc_gather_rows(h1, src)
    msg2 = _tc_messages(edge_attr, xs2, enn2_W1, enn2_b1, W2m2, b2r2, False)
    part2 = _sc_scatter16(msg2, dst)

    return _tc_final(part2, cnt, conv2_bias, bn2_gamma, bn2_beta,
                     batch.reshape(1, N_NODES), graph_features,
                     lin1_W, lin1_b, lin2_W, lin2_b)


# speedup vs baseline: 3.7086x; 3.7086x over previous
import functools

import jax
import jax.numpy as jnp
import numpy as np
from jax import lax
from jax.experimental import pallas as pl
from jax.experimental.pallas import tpu as pltpu
from jax.experimental.pallas import tpu_sc as plsc

N_NODES = 10000
N_EDGES = 160000
F = 16
N_GRAPHS = 64

NC = 2
NS = 16
NW = NC * NS
EDGES_PER_TILE = N_EDGES // NW
NODE_ROWS_PER_TILE = N_NODES // NS
SCATTER_CHUNK = 1000

_MESH = plsc.VectorSubcoreMesh(core_axis_name="c", subcore_axis_name="s")

_REP_H = np.kron(np.eye(F), np.ones((1, F))).astype(np.float32)
_REP_X = np.kron(np.ones((1, F)), np.eye(F)).astype(np.float32)


def _leaky(v):
    return jnp.where(v >= 0, v, 0.01 * v)


@functools.partial(
    pl.kernel,
    mesh=_MESH,
    out_type=jax.ShapeDtypeStruct((N_EDGES, F), jnp.float32),
    scratch_types=[
        pltpu.VMEM((EDGES_PER_TILE,), jnp.int32),
        pltpu.VMEM((EDGES_PER_TILE, F), jnp.float32),
        pltpu.SemaphoreType.DMA,
    ],
    compiler_params=pltpu.CompilerParams(use_tc_tiling_on_sc=False),
)
def _sc_gather_rows(table_hbm, idx_hbm, out_hbm, idx_v, rows_v, sem):
    wid = lax.axis_index("s") * NC + lax.axis_index("c")
    base = wid * EDGES_PER_TILE
    pltpu.sync_copy(idx_hbm.at[pl.ds(base, EDGES_PER_TILE)], idx_v)
    pltpu.async_copy(table_hbm.at[idx_v], rows_v, sem).wait()
    pltpu.sync_copy(rows_v, out_hbm.at[pl.ds(base, EDGES_PER_TILE)])


def _make_sc_scatter(width):
    nchunks = EDGES_PER_TILE // SCATTER_CHUNK

    @functools.partial(
        pl.kernel,
        mesh=_MESH,
        out_type=jax.ShapeDtypeStruct((NC, N_NODES, width), jnp.float32),
        scratch_types=[
            pltpu.VMEM((SCATTER_CHUNK, width), jnp.float32),
            pltpu.VMEM((SCATTER_CHUNK,), jnp.int32),
            pltpu.VMEM((NODE_ROWS_PER_TILE, width), jnp.float32),
            pltpu.VMEM_SHARED((N_NODES, width), jnp.float32),
        ],
        compiler_params=pltpu.CompilerParams(use_tc_tiling_on_sc=False),
    )
    def _scatter(msg_hbm, dst_hbm, out_hbm, msg_v, idx_v, row_v, acc):
        c = lax.axis_index("c")
        s = lax.axis_index("s")
        base = (c * NS + s) * EDGES_PER_TILE

        def _zero_row(i, carry):
            for j in range(width // 16):
                row_v[i, pl.ds(j * 16, 16)] = jnp.zeros((16,), jnp.float32)
            return carry

        lax.fori_loop(0, NODE_ROWS_PER_TILE, _zero_row, 0)
        pltpu.sync_copy(
            row_v, acc.at[pl.ds(s * NODE_ROWS_PER_TILE, NODE_ROWS_PER_TILE)]
        )
        plsc.subcore_barrier()
        for ch in range(nchunks):
            off = base + ch * SCATTER_CHUNK
            pltpu.sync_copy(msg_hbm.at[pl.ds(off, SCATTER_CHUNK)], msg_v)
            pltpu.sync_copy(dst_hbm.at[pl.ds(off, SCATTER_CHUNK)], idx_v)
            pltpu.sync_copy(msg_v, acc.at[idx_v], add=True)
        plsc.subcore_barrier()
        pltpu.sync_copy(
            acc.at[pl.ds(s * NODE_ROWS_PER_TILE, NODE_ROWS_PER_TILE)], row_v
        )
        pltpu.sync_copy(
            row_v,
            out_hbm.at[c, pl.ds(s * NODE_ROWS_PER_TILE, NODE_ROWS_PER_TILE)],
        )

    return _scatter


_sc_scatter32 = _make_sc_scatter(32)
_sc_scatter16 = _make_sc_scatter(16)


_EDGE_BLOCK = 2000


def _msg_body(pack_counts, ea_ref, xs_ref, w1_ref, b1_ref, rh_ref, rx_ref,
              w2m_ref, b2r_ref, out_ref):
    ea = ea_ref[...]
    xs = xs_ref[...]
    h = jnp.dot(ea, w1_ref[...], preferred_element_type=jnp.float32) + b1_ref[...]
    h = _leaky(h)
    z = (jnp.dot(h, rh_ref[...], preferred_element_type=jnp.float32)
         * jnp.dot(xs, rx_ref[...], preferred_element_type=jnp.float32))
    msg = (jnp.dot(z, w2m_ref[...], preferred_element_type=jnp.float32)
           + jnp.dot(xs, b2r_ref[...], preferred_element_type=jnp.float32))
    if pack_counts:
        out_ref[...] = jnp.concatenate([msg, jnp.ones_like(msg)], axis=1)
    else:
        out_ref[...] = msg


def _tc_messages(edge_attr, xs, W1, b1, W2m, b2r, pack_counts):
    width = 2 * F if pack_counts else F
    grid = N_EDGES // _EDGE_BLOCK
    full = lambda i: (0, 0)
    return pl.pallas_call(
        functools.partial(_msg_body, pack_counts),
        grid=(grid,),
        in_specs=[
            pl.BlockSpec((_EDGE_BLOCK, 4), lambda i: (i, 0)),
            pl.BlockSpec((_EDGE_BLOCK, F), lambda i: (i, 0)),
            pl.BlockSpec((4, F), full),
            pl.BlockSpec((1, F), full),
            pl.BlockSpec((F, F * F), full),
            pl.BlockSpec((F, F * F), full),
            pl.BlockSpec((F * F, F), full),
            pl.BlockSpec((F, F), full),
        ],
        out_specs=pl.BlockSpec((_EDGE_BLOCK, width), lambda i: (i, 0)),
        out_shape=jax.ShapeDtypeStruct((N_EDGES, width), jnp.float32),
    )(edge_attr, xs, W1, b1.reshape(1, F), _REP_H, _REP_X, W2m, b2r)


def _bn1_body(p_ref, bias_ref, g_ref, b_ref, h_ref, cnt_ref):
    p0 = p_ref[0]
    p1 = p_ref[1]
    s = p0[:, :F] + p1[:, :F]
    cnt = jnp.maximum(p0[:, F:] + p1[:, F:], 1.0)
    a = _leaky(s / cnt + bias_ref[...])
    m = jnp.mean(a, axis=0, keepdims=True)
    v = jnp.mean((a - m) ** 2, axis=0, keepdims=True)
    h_ref[...] = g_ref[...] * (a - m) * lax.rsqrt(v + 1e-5) + b_ref[...]
    cnt_ref[...] = cnt


def _tc_bn1(partials, conv_bias, gamma, beta):
    return pl.pallas_call(
        _bn1_body,
        out_shape=(
            jax.ShapeDtypeStruct((N_NODES, F), jnp.float32),
            jax.ShapeDtypeStruct((N_NODES, F), jnp.float32),
        ),
    )(partials, conv_bias.reshape(1, F), gamma.reshape(1, F), beta.reshape(1, F))


def _final_body(p_ref, cnt_ref, bias_ref, g_ref, b_ref, batch_ref, gf_ref,
                w1_ref, b1_ref, w2_ref, b2_ref, out_ref):
    s = p_ref[0] + p_ref[1]
    a = _leaky(s / cnt_ref[...] + bias_ref[...])
    m = jnp.mean(a, axis=0, keepdims=True)
    v = jnp.mean((a - m) ** 2, axis=0, keepdims=True)
    h = g_ref[...] * (a - m) * lax.rsqrt(v + 1e-5) + b_ref[...]
    gid = lax.broadcasted_iota(jnp.int32, (N_GRAPHS, N_NODES), 0)
    oh = (batch_ref[...] == gid).astype(jnp.float32)
    psum = jnp.dot(oh, h, preferred_element_type=jnp.float32)
    gcnt = jnp.maximum(jnp.sum(oh, axis=1, keepdims=True), 1.0)
    g = jnp.concatenate([psum / gcnt, gf_ref[...]], axis=1)
    g = _leaky(jnp.dot(g, w1_ref[...], preferred_element_type=jnp.float32)
               + b1_ref[...])
    g = _leaky(jnp.dot(g, w2_ref[...], preferred_element_type=jnp.float32)
               + b2_ref[...])
    out_ref[...] = g


def _tc_final(partials, cnt, conv_bias, gamma, beta, batch2d, gf,
              lin1_W, lin1_b, lin2_W, lin2_b):
    return pl.pallas_call(
        _final_body,
        out_shape=jax.ShapeDtypeStruct((N_GRAPHS, 32), jnp.float32),
    )(partials, cnt, conv_bias.reshape(1, F), gamma.reshape(1, F),
      beta.reshape(1, F), batch2d, gf, lin1_W, lin1_b.reshape(1, 64),
      lin2_W, lin2_b.reshape(1, 32))


def kernel(x, edge_index, edge_attr, graph_features, batch,
           enn1_W1, enn1_b1, enn1_W2, enn1_b2, conv1_bias, bn1_gamma, bn1_beta,
           enn2_W1, enn2_b1, enn2_W2, enn2_b2, conv2_bias, bn2_gamma, bn2_beta,
           lin1_W, lin1_b, lin2_W, lin2_b):
    src = edge_index[0]
    dst = edge_index[1]
    W2m1 = enn1_W2.reshape(F, F, F).reshape(F * F, F)
    b2r1 = enn1_b2.reshape(F, F)
    W2m2 = enn2_W2.reshape(F, F, F).reshape(F * F, F)
    b2r2 = enn2_b2.reshape(F, F)

    xs1 = _sc_gather_rows(x, src)
    msg1 = _tc_messages(edge_attr, xs1, enn1_W1, enn1_b1, W2m1, b2r1, True)
    part1 = _sc_scatter32(msg1, dst)
    h1, cnt = _tc_bn1(part1, conv1_bias, bn1_gamma, bn1_beta)

    xs2 = _sc_gather_rows(h1, src)
    msg2 = _tc_messages(edge_attr, xs2, enn2_W1, enn2_b1, W2m2, b2r2, False)
    part2 = _sc_scatter16(msg2, dst)

    return _tc_final(part2, cnt, conv2_bias, bn2_gamma, bn2_beta,
                     batch.reshape(1, N_NODES), graph_features,
                     lin1_W, lin1_b, lin2_W, lin2_b)


# speedup vs baseline: 6.3978x; 1.7251x over previous
import functools

import jax
import jax.numpy as jnp
import numpy as np
from jax import lax
from jax.experimental import pallas as pl
from jax.experimental.pallas import tpu as pltpu
from jax.experimental.pallas import tpu_sc as plsc

N_NODES = 10000
N_EDGES = 160000
F = 16
N_GRAPHS = 64

NC = 2
NS = 16
NW = NC * NS
EDGES_PER_TILE = N_EDGES // NW
NODE_ROWS_PER_TILE = N_NODES // NS
SCATTER_CHUNK = 1000
NROWS = N_NODES // 8

_MESH = plsc.VectorSubcoreMesh(core_axis_name="c", subcore_axis_name="s")
_SC_PARAMS = pltpu.CompilerParams(use_tc_tiling_on_sc=False)

_REP_H = np.kron(np.eye(F), np.ones((1, F))).astype(np.float32)
_REP_X = np.kron(np.ones((1, F)), np.eye(F)).astype(np.float32)
_EYE8 = np.eye(8, dtype=np.float32)


def _leaky(v):
    return jnp.where(v >= 0, v, 0.01 * v)


@functools.partial(
    pl.kernel,
    mesh=_MESH,
    out_type=jax.ShapeDtypeStruct((N_EDGES, F), jnp.float32),
    scratch_types=[
        pltpu.VMEM((EDGES_PER_TILE,), jnp.int32),
        pltpu.VMEM((EDGES_PER_TILE, F), jnp.float32),
        pltpu.SemaphoreType.DMA,
    ],
    compiler_params=_SC_PARAMS,
)
def _sc_gather_rows(table_hbm, idx_hbm, out_hbm, idx_v, rows_v, sem):
    wid = lax.axis_index("s") * NC + lax.axis_index("c")
    base = wid * EDGES_PER_TILE
    pltpu.sync_copy(idx_hbm.at[pl.ds(base, EDGES_PER_TILE)], idx_v)
    pltpu.async_copy(table_hbm.at[idx_v], rows_v, sem).wait()
    pltpu.sync_copy(rows_v, out_hbm.at[pl.ds(base, EDGES_PER_TILE)])


def _make_sc_scatter(with_counts):
    nchunks = EDGES_PER_TILE // SCATTER_CHUNK
    n_out = 2 if with_counts else 1
    out_type = tuple(
        jax.ShapeDtypeStruct((NC, N_NODES, F), jnp.float32)
        for _ in range(n_out)
    )
    scratch = [
        pltpu.VMEM((SCATTER_CHUNK, F), jnp.float32),
        pltpu.VMEM((SCATTER_CHUNK,), jnp.int32),
        pltpu.VMEM((NODE_ROWS_PER_TILE, F), jnp.float32),
        pltpu.VMEM_SHARED((N_NODES, F), jnp.float32),
    ]
    if with_counts:
        scratch += [
            pltpu.VMEM((SCATTER_CHUNK, F), jnp.float32),
            pltpu.VMEM_SHARED((N_NODES, F), jnp.float32),
        ]

    @functools.partial(
        pl.kernel,
        mesh=_MESH,
        out_type=out_type,
        scratch_types=scratch,
        compiler_params=_SC_PARAMS,
    )
    def _scatter(msg_hbm, dst_hbm, *refs):
        if with_counts:
            out_hbm, cnt_hbm, msg_v, idx_v, row_v, acc, ones_v, acc_c = refs
        else:
            out_hbm, msg_v, idx_v, row_v, acc = refs
        c = lax.axis_index("c")
        s = lax.axis_index("s")
        base = (c * NS + s) * EDGES_PER_TILE
        rows = pl.ds(s * NODE_ROWS_PER_TILE, NODE_ROWS_PER_TILE)

        def _fill_zero(i, carry):
            row_v[i, :] = jnp.zeros((16,), jnp.float32)
            return carry

        lax.fori_loop(0, NODE_ROWS_PER_TILE, _fill_zero, 0)
        pltpu.sync_copy(row_v, acc.at[rows])
        if with_counts:
            pltpu.sync_copy(row_v, acc_c.at[rows])

            def _fill_ones(i, carry):
                ones_v[i, :] = jnp.ones((16,), jnp.float32)
                return carry

            lax.fori_loop(0, SCATTER_CHUNK, _fill_ones, 0)
        plsc.subcore_barrier()
        for ch in range(nchunks):
            off = base + ch * SCATTER_CHUNK
            pltpu.sync_copy(msg_hbm.at[pl.ds(off, SCATTER_CHUNK)], msg_v)
            pltpu.sync_copy(dst_hbm.at[pl.ds(off, SCATTER_CHUNK)], idx_v)
            pltpu.sync_copy(msg_v, acc.at[idx_v], add=True)
            if with_counts:
                pltpu.sync_copy(ones_v, acc_c.at[idx_v], add=True)
        plsc.subcore_barrier()
        pltpu.sync_copy(acc.at[rows], row_v)
        pltpu.sync_copy(row_v, out_hbm.at[c, rows])
        if with_counts:
            pltpu.sync_copy(acc_c.at[rows], row_v)
            pltpu.sync_copy(row_v, cnt_hbm.at[c, rows])

    return _scatter


_sc_scatter_counts = _make_sc_scatter(True)
_sc_scatter_plain = _make_sc_scatter(False)


_EDGE_BLOCK = 6400
_EB8 = _EDGE_BLOCK // 8


def _msg_body(ea_ref, xs_ref, w1b_ref, b1t_ref, rh_ref, rx_ref,
              w2m_ref, b2r_ref, out_ref):
    h8 = _leaky(jnp.dot(ea_ref[...], w1b_ref[...],
                        preferred_element_type=jnp.float32) + b1t_ref[...])
    xs8 = xs_ref[...]
    for j in range(8):
        sl = pl.ds(j * F, F)
        h = h8[:, j * F:(j + 1) * F]
        xs = xs8[:, j * F:(j + 1) * F]
        z = (jnp.dot(h, rh_ref[...], preferred_element_type=jnp.float32)
             * jnp.dot(xs, rx_ref[...], preferred_element_type=jnp.float32))
        out_ref[:, sl] = (
            jnp.dot(z, w2m_ref[...], preferred_element_type=jnp.float32)
            + jnp.dot(xs, b2r_ref[...], preferred_element_type=jnp.float32))


def _tc_messages(ea32, xs128, W1b, b1t, W2m, b2r):
    grid = N_EDGES // _EDGE_BLOCK
    full = lambda i: (0, 0)
    return pl.pallas_call(
        _msg_body,
        grid=(grid,),
        in_specs=[
            pl.BlockSpec((_EB8, 32), lambda i: (i, 0)),
            pl.BlockSpec((_EB8, 128), lambda i: (i, 0)),
            pl.BlockSpec((32, 128), full),
            pl.BlockSpec((1, 128), full),
            pl.BlockSpec((F, F * F), full),
            pl.BlockSpec((F, F * F), full),
            pl.BlockSpec((F * F, F), full),
            pl.BlockSpec((F, F), full),
        ],
        out_specs=pl.BlockSpec((_EB8, 128), lambda i: (i, 0)),
        out_shape=jax.ShapeDtypeStruct((N_EDGES // 8, 128), jnp.float32),
    )(ea32, xs128, W1b, b1t, _REP_H, _REP_X, W2m, b2r)


def _fold16(v128):
    t = v128[:, 0:F]
    for j in range(1, 8):
        t = t + v128[:, j * F:(j + 1) * F]
    return t


def _tile8(v16):
    return jnp.concatenate([v16] * 8, axis=1)


def _bn_stats(a128):
    m = _tile8(_fold16(jnp.sum(a128, axis=0, keepdims=True)) / N_NODES)
    d = a128 - m
    v = _tile8(_fold16(jnp.sum(d * d, axis=0, keepdims=True)) / N_NODES)
    return m, v


def _bn1_body(p_ref, pc_ref, bias_ref, g_ref, b_ref, h_ref, cnt_ref):
    p = p_ref[...]
    pc = pc_ref[...]
    s = p[:NROWS] + p[NROWS:]
    cnt = jnp.maximum(pc[:NROWS] + pc[NROWS:], 1.0)
    a = _leaky(s / cnt + bias_ref[...])
    m, v = _bn_stats(a)
    h_ref[...] = g_ref[...] * (a - m) * lax.rsqrt(v + 1e-5) + b_ref[...]
    cnt_ref[...] = cnt


def _tc_bn1(p128, pc128, bias_t, gamma_t, beta_t):
    return pl.pallas_call(
        _bn1_body,
        out_shape=(
            jax.ShapeDtypeStruct((NROWS, 128), jnp.float32),
            jax.ShapeDtypeStruct((NROWS, 128), jnp.float32),
        ),
    )(p128, pc128, bias_t, gamma_t, beta_t)


def _final_body(p_ref, cnt_ref, bias_ref, g_ref, b_ref, batch8_ref, gf_ref,
                w1_ref, b1_ref, w2_ref, b2_ref, out_ref):
    p = p_ref[...]
    s = p[:NROWS] + p[NROWS:]
    a = _leaky(s / cnt_ref[...] + bias_ref[...])
    m, v = _bn_stats(a)
    h = g_ref[...] * (a - m) * lax.rsqrt(v + 1e-5) + b_ref[...]
    gid = lax.broadcasted_iota(jnp.int32, (N_GRAPHS, NROWS), 0)
    psum = jnp.zeros((N_GRAPHS, F), jnp.float32)
    gcnt = jnp.zeros((N_GRAPHS, 1), jnp.float32)
    for j in range(8):
        oh = (batch8_ref[j:j + 1, :] == gid).astype(jnp.float32)
        psum = psum + jnp.dot(oh, h[:, j * F:(j + 1) * F],
                              preferred_element_type=jnp.float32)
        gcnt = gcnt + jnp.sum(oh, axis=1, keepdims=True)
    pooled = psum / jnp.maximum(gcnt, 1.0)
    g = jnp.concatenate([pooled, gf_ref[...]], axis=1)
    g = _leaky(jnp.dot(g, w1_ref[...], preferred_element_type=jnp.float32)
               + b1_ref[...])
    g = _leaky(jnp.dot(g, w2_ref[...], preferred_element_type=jnp.float32)
               + b2_ref[...])
    out_ref[...] = g


def _tc_final(p128, cnt128, bias_t, gamma_t, beta_t, batch8, gf,
              lin1_W, lin1_b, lin2_W, lin2_b):
    return pl.pallas_call(
        _final_body,
        out_shape=jax.ShapeDtypeStruct((N_GRAPHS, 32), jnp.float32),
    )(p128, cnt128, bias_t, gamma_t, beta_t, batch8, gf,
      lin1_W, lin1_b.reshape(1, 64), lin2_W, lin2_b.reshape(1, 32))


def kernel(x, edge_index, edge_attr, graph_features, batch,
           enn1_W1, enn1_b1, enn1_W2, enn1_b2, conv1_bias, bn1_gamma, bn1_beta,
           enn2_W1, enn2_b1, enn2_W2, enn2_b2, conv2_bias, bn2_gamma, bn2_beta,
           lin1_W, lin1_b, lin2_W, lin2_b):
    src = edge_index[0]
    dst = edge_index[1]
    ea32 = edge_attr.reshape(N_EDGES // 8, 32)
    eye8 = jnp.asarray(_EYE8)
    W1b1 = jnp.kron(eye8, enn1_W1)
    W1b2 = jnp.kron(eye8, enn2_W1)
    b1t1 = jnp.tile(enn1_b1, 8).reshape(1, 128)
    b1t2 = jnp.tile(enn2_b1, 8).reshape(1, 128)
    W2m1 = enn1_W2.reshape(F, F, F).reshape(F * F, F)
    b2r1 = enn1_b2.reshape(F, F)
    W2m2 = enn2_W2.reshape(F, F, F).reshape(F * F, F)
    b2r2 = enn2_b2.reshape(F, F)
    bias1_t = jnp.tile(conv1_bias, 8).reshape(1, 128)
    bias2_t = jnp.tile(conv2_bias, 8).reshape(1, 128)
    g1_t = jnp.tile(bn1_gamma, 8).reshape(1, 128)
    b1_t = jnp.tile(bn1_beta, 8).reshape(1, 128)
    g2_t = jnp.tile(bn2_gamma, 8).reshape(1, 128)
    b2_t = jnp.tile(bn2_beta, 8).reshape(1, 128)
    batch8 = batch.reshape(NROWS, 8).T

    xs1 = _sc_gather_rows(x, src)
    msg1 = _tc_messages(ea32, xs1.reshape(N_EDGES // 8, 128),
                        W1b1, b1t1, W2m1, b2r1)
    part1, part1c = _sc_scatter_counts(msg1.reshape(N_EDGES, F), dst)
    h1_128, cnt128 = _tc_bn1(part1.reshape(NC * NROWS, 128),
                             part1c.reshape(NC * NROWS, 128),
                             bias1_t, g1_t, b1_t)

    xs2 = _sc_gather_rows(h1_128.reshape(N_NODES, F), src)
    msg2 = _tc_messages(ea32, xs2.reshape(N_EDGES // 8, 128),
                        W1b2, b1t2, W2m2, b2r2)
    (part2,) = _sc_scatter_plain(msg2.reshape(N_EDGES, F), dst)

    return _tc_final(part2.reshape(NC * NROWS, 128), cnt128,
                     bias2_t, g2_t, b2_t, batch8, graph_features,
                     lin1_W, lin1_b, lin2_W, lin2_b)
